# baseline (speedup 1.0000x reference)
import functools

import jax
import jax.numpy as jnp
from jax import lax
from jax.experimental import pallas as pl
from jax.experimental.pallas import tpu as pltpu
from jax.experimental.pallas import tpu_sc as plsc

D = 32
B = 16384
N = 1000000
NC, NS, L = 2, 16, 16
NW = NC * NS
BPW = B // NW
G = BPW // L

A0 = 655360
RA = A0 // 128
CPW = A0 // NW
CB = 512
NF = CPW // CB
TCB = 131072
NTC = 3
RB = (NTC * TCB) // 128
ROWS = RA + RB


def _dot_body(u_ref, t_ref, out_ref):
    x = t_ref[...]
    u = u_ref[...]
    y = jax.lax.dot_general(
        u, x, (((1,), (0,)), ((), ())),
        preferred_element_type=jnp.float32,
    )
    out_ref[...] = y[0]


_dot_call = pl.pallas_call(
    _dot_body,
    grid=(NTC,),
    in_specs=[
        pl.BlockSpec((8, D), lambda i: (0, 0)),
        pl.BlockSpec((D, TCB), lambda i: (0, i + A0 // TCB)),
    ],
    out_specs=pl.BlockSpec((TCB,), lambda i: (i,)),
    out_shape=jax.ShapeDtypeStruct((NTC * TCB,), jnp.float32),
)

_mesh = plsc.VectorSubcoreMesh(
    core_axis_name="c", subcore_axis_name="s", num_cores=NC, num_subcores=NS
)
_sc_params = pltpu.CompilerParams(needs_layout_passes=False)


@functools.partial(
    pl.kernel,
    out_type=jax.ShapeDtypeStruct((A0,), jnp.float32),
    mesh=_mesh,
    compiler_params=_sc_params,
    scratch_types=[
        pltpu.VMEM((D, CB), jnp.float32),
        pltpu.VMEM((D, CB), jnp.float32),
        pltpu.VMEM((D, L), jnp.float32),
        pltpu.VMEM((CPW,), jnp.float32),
        pltpu.SemaphoreType.DMA,
        pltpu.SemaphoreType.DMA,
    ],
)
def _sc_matvec(userb_hbm, tablet_hbm, out_hbm, buf0, buf1, ub_v, out_v,
               sem0, sem1):
    wid = lax.axis_index("s") * NC + lax.axis_index("c")
    c0 = wid * CPW
    pltpu.sync_copy(userb_hbm, ub_v)

    def slab(buf, sem, f):
        col = pl.multiple_of(c0 + f * CB, 128)
        return pltpu.async_copy(tablet_hbm.at[:, pl.ds(col, CB)], buf, sem)

    def compute(buf, f):
        def group(g, carry):
            acc = jnp.zeros((L,), jnp.float32)
            for j in range(D):
                acc = acc + buf[j, pl.ds(g * L, L)] * ub_v[j]
            out_v[pl.ds(f * CB + g * L, L)] = acc
            return carry

        lax.fori_loop(0, CB // L, group, 0)

    slab(buf0, sem0, 0)

    def outer(o, carry):
        f = 2 * o
        slab(buf1, sem1, f + 1)
        pltpu.make_async_copy(tablet_hbm.at[:, pl.ds(0, CB)], buf0,
                              sem0).wait()
        compute(buf0, f)

        @pl.when(o < NF // 2 - 1)
        def _():
            slab(buf0, sem0, f + 2)

        pltpu.make_async_copy(tablet_hbm.at[:, pl.ds(0, CB)], buf1,
                              sem1).wait()
        compute(buf1, f + 1)
        return carry

    lax.fori_loop(0, NF // 2, outer, 0)
    pltpu.sync_copy(out_v, out_hbm.at[pl.ds(c0, CPW)])


@functools.partial(
    pl.kernel,
    out_type=jax.ShapeDtypeStruct((B,), jnp.float32),
    mesh=_mesh,
    compiler_params=_sc_params,
    scratch_types=[
        pltpu.VMEM((BPW,), jnp.int32),
        pltpu.VMEM((BPW,), jnp.int32),
        pltpu.VMEM((BPW, 128), jnp.float32),
        pltpu.VMEM((BPW,), jnp.float32),
        pltpu.SemaphoreType.DMA,
    ],
)
def _sc_gather(idx_hbm, logits_hbm, out_hbm, idx_v, row_v, rows_v, out_v,
               sem):
    wid = lax.axis_index("s") * NC + lax.axis_index("c")
    base = wid * BPW
    pltpu.sync_copy(idx_hbm.at[pl.ds(base, BPW)], idx_v)

    def split(g, carry):
        v = idx_v[pl.ds(g * L, L)]
        row_v[pl.ds(g * L, L)] = lax.shift_right_logical(v, 7)
        return carry

    lax.fori_loop(0, G, split, 0)
    pltpu.async_copy(logits_hbm.at[row_v], rows_v, sem).wait()
    lane = lax.iota(jnp.int32, L)

    def group(g, carry):
        col = jnp.bitwise_and(idx_v[pl.ds(g * L, L)], 127)
        x = plsc.load_gather(rows_v, [g * L + lane, col])
        out_v[pl.ds(g * L, L)] = 1.0 / (1.0 + jnp.exp(-x))
        return carry

    lax.fori_loop(0, G, group, 0)
    pltpu.sync_copy(out_v, out_hbm.at[pl.ds(base, BPW)])


def kernel(item_indices, embedding_user, embedding_item):
    tablet = embedding_item.T
    u8 = jnp.broadcast_to(embedding_user.reshape(1, D), (8, D))
    userb = jnp.broadcast_to(embedding_user.reshape(D, 1), (D, L))
    logits_a = _sc_matvec(userb, tablet).reshape(RA, 128)
    logits_b = _dot_call(u8, tablet).reshape(RB, 128)
    logits = jnp.concatenate([logits_a, logits_b], axis=0)
    return _sc_gather(item_indices, logits)


# speedup vs baseline: 1.0016x; 1.0016x over previous
import functools

import jax
import jax.numpy as jnp
from jax import lax
from jax.experimental import pallas as pl
from jax.experimental.pallas import tpu as pltpu
from jax.experimental.pallas import tpu_sc as plsc

D = 32
B = 16384
N = 1000000
NC, NS, L = 2, 16, 16
NW = NC * NS
BPW = B // NW
G = BPW // L

A0 = 655360
RA = A0 // 128
CPW = A0 // NW
CB = 512
NF = CPW // CB
TCB = 131072
NTC = 3
RB = (NTC * TCB) // 128
ROWS = RA + RB


def _dot_body(u_ref, t_ref, out_ref):
    x = t_ref[...]
    u = u_ref[...]
    y = jax.lax.dot_general(
        u, x, (((1,), (0,)), ((), ())),
        preferred_element_type=jnp.float32,
    )
    out_ref[...] = y[0]


_dot_call = pl.pallas_call(
    _dot_body,
    grid=(NTC,),
    in_specs=[
        pl.BlockSpec((8, D), lambda i: (0, 0)),
        pl.BlockSpec((D, TCB), lambda i: (0, i + A0 // TCB)),
    ],
    out_specs=pl.BlockSpec((TCB,), lambda i: (i,)),
    out_shape=jax.ShapeDtypeStruct((NTC * TCB,), jnp.float32),
)

_mesh = plsc.VectorSubcoreMesh(
    core_axis_name="c", subcore_axis_name="s", num_cores=NC, num_subcores=NS
)
_sc_params = pltpu.CompilerParams(needs_layout_passes=False)


@functools.partial(
    pl.kernel,
    out_type=jax.ShapeDtypeStruct((A0,), jnp.float32),
    mesh=_mesh,
    compiler_params=_sc_params,
    scratch_types=[
        pltpu.VMEM((D, CB), jnp.float32),
        pltpu.VMEM((D, CB), jnp.float32),
        pltpu.VMEM((D, L), jnp.float32),
        pltpu.VMEM((CPW,), jnp.float32),
        pltpu.SemaphoreType.DMA,
        pltpu.SemaphoreType.DMA,
    ],
)
def _sc_matvec(userb_hbm, tablet_hbm, out_hbm, buf0, buf1, ub_v, out_v,
               sem0, sem1):
    wid = lax.axis_index("s") * NC + lax.axis_index("c")
    c0 = wid * CPW
    pltpu.sync_copy(userb_hbm, ub_v)

    def slab(buf, sem, f):
        col = pl.multiple_of(c0 + f * CB, 128)
        return pltpu.async_copy(tablet_hbm.at[:, pl.ds(col, CB)], buf, sem)

    def compute(buf, f):
        def group(g, carry):
            acc = jnp.zeros((L,), jnp.float32)
            for j in range(D):
                acc = acc + buf[j, pl.ds(g * L, L)] * ub_v[j]
            out_v[pl.ds(f * CB + g * L, L)] = acc
            return carry

        lax.fori_loop(0, CB // L, group, 0)

    slab(buf0, sem0, 0)

    def outer(o, carry):
        f = 2 * o
        slab(buf1, sem1, f + 1)
        pltpu.make_async_copy(tablet_hbm.at[:, pl.ds(0, CB)], buf0,
                              sem0).wait()
        compute(buf0, f)

        @pl.when(o < NF // 2 - 1)
        def _():
            slab(buf0, sem0, f + 2)

        pltpu.make_async_copy(tablet_hbm.at[:, pl.ds(0, CB)], buf1,
                              sem1).wait()
        compute(buf1, f + 1)
        return carry

    lax.fori_loop(0, NF // 2, outer, 0)
    pltpu.sync_copy(out_v, out_hbm.at[pl.ds(c0, CPW)])


@functools.partial(
    pl.kernel,
    out_type=jax.ShapeDtypeStruct((B,), jnp.float32),
    mesh=_mesh,
    compiler_params=_sc_params,
    scratch_types=[
        pltpu.VMEM((BPW,), jnp.int32),
        pltpu.VMEM((BPW,), jnp.int32),
        pltpu.VMEM((BPW, 128), jnp.float32),
        pltpu.VMEM((BPW,), jnp.float32),
        pltpu.SemaphoreType.DMA,
    ],
)
def _sc_gather(idx_hbm, logits_hbm, out_hbm, idx_v, row_v, rows_v, out_v,
               sem):
    wid = lax.axis_index("s") * NC + lax.axis_index("c")
    base = wid * BPW
    pltpu.sync_copy(idx_hbm.at[pl.ds(base, BPW)], idx_v)

    def split(g, carry):
        v = idx_v[pl.ds(g * L, L)]
        row_v[pl.ds(g * L, L)] = lax.shift_right_logical(v, 7)
        return carry

    lax.fori_loop(0, G, split, 0)
    pltpu.async_copy(logits_hbm.at[row_v], rows_v, sem).wait()
    lane = lax.iota(jnp.int32, L)

    def group(g, carry):
        col = jnp.bitwise_and(idx_v[pl.ds(g * L, L)], 127)
        x = plsc.load_gather(rows_v, [g * L + lane, col])
        out_v[pl.ds(g * L, L)] = 1.0 / (1.0 + jnp.exp(-x))
        return carry

    lax.fori_loop(0, G, group, 0)
    pltpu.sync_copy(out_v, out_hbm.at[pl.ds(base, BPW)])


def kernel(item_indices, embedding_user, embedding_item):
    tablet = embedding_item.T
    u8 = jnp.broadcast_to(embedding_user.reshape(1, D), (8, D))
    userb = jnp.broadcast_to(embedding_user.reshape(D, 1), (D, L))
    logits_b = _dot_call(u8, tablet).reshape(RB, 128)
    logits_a = _sc_matvec(userb, tablet).reshape(RA, 128)
    logits = jnp.concatenate([logits_a, logits_b], axis=0)
    return _sc_gather(item_indices, logits)


# speedup vs baseline: 1.4345x; 1.4321x over previous
import functools

import jax
import jax.numpy as jnp
from jax import lax
from jax.experimental import pallas as pl
from jax.experimental.pallas import tpu as pltpu
from jax.experimental.pallas import tpu_sc as plsc

D = 32
B = 16384
N = 1000000
NPAD = 1000064
ROWS = NPAD // 128
CHUNK = 131072
NCHUNK = -(-NPAD // CHUNK)
NC, NS, L = 2, 16, 16
NW = NC * NS
BPW = B // NW
G = BPW // L


def _dot_body(u_ref, t_ref, out_ref):
    x = t_ref[...]
    u = u_ref[...]
    y = jnp.sum(x * u[0].reshape(D, 1), axis=0)
    out_ref[...] = y


_dot_call = pl.pallas_call(
    _dot_body,
    grid=(NCHUNK,),
    in_specs=[
        pl.BlockSpec((8, D), lambda i: (0, 0)),
        pl.BlockSpec((D, CHUNK), lambda i: (0, i)),
    ],
    out_specs=pl.BlockSpec((CHUNK,), lambda i: (i,)),
    out_shape=jax.ShapeDtypeStruct((NPAD,), jnp.float32),
)

_mesh = plsc.VectorSubcoreMesh(
    core_axis_name="c", subcore_axis_name="s", num_cores=NC, num_subcores=NS
)


@functools.partial(
    pl.kernel,
    out_type=jax.ShapeDtypeStruct((B,), jnp.float32),
    mesh=_mesh,
    compiler_params=pltpu.CompilerParams(needs_layout_passes=False),
    scratch_types=[
        pltpu.VMEM((BPW,), jnp.int32),
        pltpu.VMEM((BPW,), jnp.int32),
        pltpu.VMEM((BPW, 128), jnp.float32),
        pltpu.VMEM((BPW,), jnp.float32),
        pltpu.SemaphoreType.DMA,
    ],
)
def _sc_kernel(idx_hbm, logits_hbm, out_hbm, idx_v, row_v, rows_v, out_v,
               sem):
    wid = lax.axis_index("s") * NC + lax.axis_index("c")
    base = wid * BPW
    pltpu.sync_copy(idx_hbm.at[pl.ds(base, BPW)], idx_v)

    def split(g, carry):
        v = idx_v[pl.ds(g * L, L)]
        row_v[pl.ds(g * L, L)] = lax.shift_right_logical(v, 7)
        return carry

    lax.fori_loop(0, G, split, 0)
    pltpu.async_copy(logits_hbm.at[row_v], rows_v, sem).wait()
    lane = lax.iota(jnp.int32, L)

    def group(g, carry):
        col = jnp.bitwise_and(idx_v[pl.ds(g * L, L)], 127)
        x = plsc.load_gather(rows_v, [g * L + lane, col])
        out_v[pl.ds(g * L, L)] = 1.0 / (1.0 + jnp.exp(-x))
        return carry

    lax.fori_loop(0, G, group, 0)
    pltpu.sync_copy(out_v, out_hbm.at[pl.ds(base, BPW)])


def kernel(item_indices, embedding_user, embedding_item):
    u8 = jnp.broadcast_to(embedding_user.reshape(1, D), (8, D))
    logits = _dot_call(u8, embedding_item.T).reshape(ROWS, 128)
    return _sc_kernel(item_indices, logits)
